# baseline (speedup 1.0000x reference)
import numpy as np
import jax
import jax.numpy as jnp
from jax.experimental import pallas as pl
from jax.experimental.pallas import tpu as pltpu

_C, _D, _H, _W = 4, 128, 224, 224
_NS = 64
_DBLK = 16
_NK = _D // _DBLK
_SBLK = _NS // _NK

_AX = np.linspace(0, _D - 1, _NS).astype(np.int32)
_SG = np.linspace(0, _W - 1, _NS).astype(np.int32)
_CO = np.linspace(0, _H - 1, _NS).astype(np.int32)
assert all(_AX[k * _SBLK + j] // _DBLK == k
           for k in range(_NK) for j in range(_SBLK))
_AX_LOCAL = _AX.reshape(_NK, _SBLK) - (np.arange(_NK) * _DBLK)[:, None]


def _onehot(idx, n):
    m = np.zeros((_NS, n), np.float32)
    m[np.arange(_NS), idx] = 1.0
    return jnp.asarray(m)


def _body(oh_co_ref, oh_sg_ref, vol_ref, ax_ref, sag_ref, cor_ref):
    k = pl.program_id(1)
    oh_co = oh_co_ref[...]
    oh_sg = oh_sg_ref[...]
    for p in range(_DBLK):
        t = vol_ref[0, p]
        cor_ref[0, :, p, :] = jax.lax.dot_general(
            oh_co, t, (((1,), (0,)), ((), ())),
            preferred_element_type=jnp.float32,
            precision=jax.lax.Precision.HIGHEST)
        sag_ref[0, :, p, :] = jax.lax.dot_general(
            oh_sg, t, (((1,), (1,)), ((), ())),
            preferred_element_type=jnp.float32,
            precision=jax.lax.Precision.HIGHEST)
    for j in range(_SBLK):
        if np.all(_AX_LOCAL[:, j] == _AX_LOCAL[0, j]):
            ax_ref[0, j] = vol_ref[0, int(_AX_LOCAL[0, j])]
        else:
            lj = jnp.where(k == _NK - 1, int(_AX_LOCAL[-1, j]),
                           int(_AX_LOCAL[0, j]))
            ax_ref[0, j] = vol_ref[0, lj]


@jax.jit
def kernel(volume):
    oh_co = _onehot(_CO, _H)
    oh_sg = _onehot(_SG, _W)
    out = pl.pallas_call(
        _body,
        grid=(_C, _NK),
        in_specs=[
            pl.BlockSpec((_NS, _H), lambda c, k: (0, 0)),
            pl.BlockSpec((_NS, _W), lambda c, k: (0, 0)),
            pl.BlockSpec((1, _DBLK, _H, _W), lambda c, k: (c, k, 0, 0)),
        ],
        out_specs=[
            pl.BlockSpec((1, _SBLK, _H, _W), lambda c, k: (c, k, 0, 0)),
            pl.BlockSpec((1, _NS, _DBLK, _H), lambda c, k: (c, 0, k, 0)),
            pl.BlockSpec((1, _NS, _DBLK, _W), lambda c, k: (c, 0, k, 0)),
        ],
        out_shape=[
            jax.ShapeDtypeStruct((_C, _NS, _H, _W), jnp.float32),
            jax.ShapeDtypeStruct((_C, _NS, _D, _H), jnp.float32),
            jax.ShapeDtypeStruct((_C, _NS, _D, _W), jnp.float32),
        ],
        compiler_params=pltpu.CompilerParams(
            dimension_semantics=("parallel", "parallel")),
    )(oh_co, oh_sg, volume)
    axial, sagittal, coronal = out
    return (axial, sagittal, coronal)
